# baseline (speedup 1.0000x reference)
import functools

import jax
import jax.numpy as jnp
from jax import lax
from jax.experimental import pallas as pl
from jax.experimental.pallas import tpu as pltpu
from jax.experimental.pallas import tpu_sc as plsc

_VOCAB = 1000000
_EMB = 64
_HID = 128
_OUT = 128
_BATCH = 16384
_BN_EPS = 1e-3

_B_TOT = 3 * _BATCH

_NC = 2
_NS = 16
_NW = _NC * _NS
_B_PER_W = _B_TOT // _NW
_CHUNK = 128
_N_CHUNK = _B_PER_W // _CHUNK

_MLP_BLK = 2048


def _sc_gather(table, idx3):
    mesh = plsc.VectorSubcoreMesh(core_axis_name="c", subcore_axis_name="s")

    @functools.partial(
        pl.kernel,
        mesh=mesh,
        out_type=jax.ShapeDtypeStruct((_B_TOT, _EMB), jnp.float32),
        scratch_types=[
            pltpu.VMEM((_N_CHUNK, _CHUNK), jnp.int32),
            pltpu.VMEM((_B_PER_W, _EMB), jnp.float32),
            pltpu.SemaphoreType.DMA,
        ],
        compiler_params=pltpu.CompilerParams(use_tc_tiling_on_sc=False),
    )
    def gather_kernel(table_hbm, idx_hbm, out_hbm, idx_v, rows_v, sem):
        wid = lax.axis_index("s") * _NC + lax.axis_index("c")
        base = wid * _B_PER_W
        pltpu.sync_copy(idx_hbm.at[wid], idx_v)
        copies = []
        for j in range(_N_CHUNK):
            copies.append(
                pltpu.async_copy(
                    table_hbm.at[idx_v.at[j]],
                    rows_v.at[pl.ds(j * _CHUNK, _CHUNK)],
                    sem,
                )
            )
        for c in copies:
            c.wait()
        pltpu.sync_copy(rows_v, out_hbm.at[pl.ds(base, _B_PER_W)])

    return gather_kernel(table, idx3)


def _mlp_body(x_ref, w1_ref, b1_ref, s_ref, t_ref, w2_ref, b2_ref, o_ref):
    h = jnp.dot(x_ref[...], w1_ref[...], preferred_element_type=jnp.float32)
    h = jnp.maximum(h + b1_ref[...], 0.0)
    h = h * s_ref[...] + t_ref[...]
    o_ref[...] = (
        jnp.dot(h, w2_ref[...], preferred_element_type=jnp.float32) + b2_ref[...]
    )


def _tc_mlp(x, W1, b1, scale, shift, W2, b2):
    grid = (_B_TOT // _MLP_BLK,)
    return pl.pallas_call(
        _mlp_body,
        grid=grid,
        in_specs=[
            pl.BlockSpec((_MLP_BLK, _EMB), lambda i: (i, 0)),
            pl.BlockSpec((_EMB, _HID), lambda i: (0, 0)),
            pl.BlockSpec((1, _HID), lambda i: (0, 0)),
            pl.BlockSpec((1, _HID), lambda i: (0, 0)),
            pl.BlockSpec((1, _HID), lambda i: (0, 0)),
            pl.BlockSpec((_HID, _OUT), lambda i: (0, 0)),
            pl.BlockSpec((1, _OUT), lambda i: (0, 0)),
        ],
        out_specs=pl.BlockSpec((_MLP_BLK, _OUT), lambda i: (i, 0)),
        out_shape=jax.ShapeDtypeStruct((_B_TOT, _OUT), jnp.float32),
    )(x, W1, b1, scale, shift, W2, b2)


def kernel(anchor, positive, negative, table, W1, b1, gamma, beta,
           moving_mean, moving_var, W2, b2):
    idx = jnp.concatenate([anchor, positive, negative]).astype(jnp.int32)
    idx3 = idx.reshape(_NW, _N_CHUNK, _CHUNK)
    rows = _sc_gather(table, idx3)

    scale = gamma * lax.rsqrt(moving_var + _BN_EPS)
    shift = beta - moving_mean * scale
    out = _tc_mlp(
        rows,
        W1,
        b1.reshape(1, _HID),
        scale.reshape(1, _HID),
        shift.reshape(1, _HID),
        W2,
        b2.reshape(1, _OUT),
    )
    return (out[:_BATCH], out[_BATCH:2 * _BATCH], out[2 * _BATCH:])


# speedup vs baseline: 1.6883x; 1.6883x over previous
import functools

import jax
import jax.numpy as jnp
from jax import lax
from jax.experimental import pallas as pl
from jax.experimental.pallas import tpu as pltpu
from jax.experimental.pallas import tpu_sc as plsc

_VOCAB = 1000000
_EMB = 64
_HID = 128
_OUT = 128
_BATCH = 16384
_BN_EPS = 1e-3

_B_TOT = 3 * _BATCH

_NC = 2
_NS = 16
_NW = _NC * _NS
_B_PER_W = _B_TOT // _NW
_CHUNK = 32
_N_CHUNK = _B_PER_W // _CHUNK
_TROWS = 8

_MLP_BLK = 2048


def _sc_gather(table3, idx_tile, idx_row):
    mesh = plsc.VectorSubcoreMesh(core_axis_name="c", subcore_axis_name="s")

    @functools.partial(
        pl.kernel,
        mesh=mesh,
        out_type=jax.ShapeDtypeStruct((_B_TOT, _EMB), jnp.float32),
        scratch_types=[
            pltpu.VMEM((_B_PER_W,), jnp.int32),
            pltpu.VMEM((_B_PER_W,), jnp.int32),
            pltpu.VMEM((2, _CHUNK, _TROWS, _EMB), jnp.float32),
            pltpu.VMEM((2, _CHUNK, _EMB), jnp.float32),
            pltpu.SemaphoreType.DMA((2,)),
            pltpu.SemaphoreType.DMA((2,)),
        ],
        compiler_params=pltpu.CompilerParams(needs_layout_passes=False),
    )
    def gather_kernel(table_hbm, idxt_hbm, idxr_hbm, out_hbm,
                      idxt_v, idxr_v, tiles_v, rows_v, gsem, wsem):
        wid = lax.axis_index("s") * _NC + lax.axis_index("c")
        base = wid * _B_PER_W
        pltpu.sync_copy(idxt_hbm.at[wid], idxt_v)
        pltpu.sync_copy(idxr_hbm.at[wid], idxr_v)

        lane = lax.iota(jnp.int32, 16)

        def fire(j, b):
            for g in range(_CHUNK // 16):
                tv = idxt_v[pl.ds(j * _CHUNK + g * 16, 16)]
                for l in range(16):
                    pltpu.async_copy(
                        table_hbm.at[tv[l]],
                        tiles_v.at[b].at[g * 16 + l],
                        gsem.at[b])

        fire(0, 0)

        def chunk_step(j, b):
            nxt = j + 1

            @pl.when(nxt < _N_CHUNK)
            def _():
                fire(nxt, 1 - b)

            pltpu.make_async_copy(
                table_hbm.at[pl.ds(0, _CHUNK)], tiles_v.at[b],
                gsem.at[b]).wait()

            @pl.when(j >= 2)
            def _():
                pltpu.make_async_copy(
                    rows_v.at[b], out_hbm.at[pl.ds(0, _CHUNK)],
                    wsem.at[b]).wait()

            for g in range(_CHUNK // 16):
                i0 = g * 16 + lane
                r = idxr_v[pl.ds(j * _CHUNK + g * 16, 16)]
                for c in range(_EMB):
                    cc = jnp.full((16,), c, jnp.int32)
                    val = plsc.load_gather(tiles_v.at[b], [i0, r, cc])
                    plsc.store_scatter(rows_v.at[b], [i0, cc], val)

            pltpu.async_copy(
                rows_v.at[b],
                out_hbm.at[pl.ds(base + j * _CHUNK, _CHUNK)],
                wsem.at[b])

        def loop_body(j2, carry):
            chunk_step(j2 * 2, 0)
            chunk_step(j2 * 2 + 1, 1)
            return carry

        lax.fori_loop(0, _N_CHUNK // 2, loop_body, 0)

        for b in range(2):
            pltpu.make_async_copy(
                rows_v.at[b], out_hbm.at[pl.ds(0, _CHUNK)], wsem.at[b]).wait()

    return gather_kernel(table3, idx_tile, idx_row)


def _mlp_body(x_ref, w1_ref, b1_ref, s_ref, t_ref, w2_ref, b2_ref, o_ref):
    h = jnp.dot(x_ref[...], w1_ref[...], preferred_element_type=jnp.float32)
    h = jnp.maximum(h + b1_ref[...], 0.0)
    h = h * s_ref[...] + t_ref[...]
    o_ref[...] = (
        jnp.dot(h, w2_ref[...], preferred_element_type=jnp.float32) + b2_ref[...]
    )


def _tc_mlp(x, W1, b1, scale, shift, W2, b2):
    grid = (_B_TOT // _MLP_BLK,)
    return pl.pallas_call(
        _mlp_body,
        grid=grid,
        in_specs=[
            pl.BlockSpec((_MLP_BLK, _EMB), lambda i: (i, 0)),
            pl.BlockSpec((_EMB, _HID), lambda i: (0, 0)),
            pl.BlockSpec((1, _HID), lambda i: (0, 0)),
            pl.BlockSpec((1, _HID), lambda i: (0, 0)),
            pl.BlockSpec((1, _HID), lambda i: (0, 0)),
            pl.BlockSpec((_HID, _OUT), lambda i: (0, 0)),
            pl.BlockSpec((1, _OUT), lambda i: (0, 0)),
        ],
        out_specs=pl.BlockSpec((_MLP_BLK, _OUT), lambda i: (i, 0)),
        out_shape=jax.ShapeDtypeStruct((_B_TOT, _OUT), jnp.float32),
    )(x, W1, b1, scale, shift, W2, b2)


def kernel(anchor, positive, negative, table, W1, b1, gamma, beta,
           moving_mean, moving_var, W2, b2):
    idx = jnp.concatenate([anchor, positive, negative]).astype(jnp.int32)
    idx_tile = (idx >> 3).reshape(_NW, _B_PER_W)
    idx_row = (idx & 7).reshape(_NW, _B_PER_W)
    table3 = table.reshape(_VOCAB // _TROWS, _TROWS, _EMB)
    rows = _sc_gather(table3, idx_tile, idx_row)

    scale = gamma * lax.rsqrt(moving_var + _BN_EPS)
    shift = beta - moving_mean * scale
    out = _tc_mlp(
        rows,
        W1,
        b1.reshape(1, _HID),
        scale.reshape(1, _HID),
        shift.reshape(1, _HID),
        W2,
        b2.reshape(1, _OUT),
    )
    return (out[:_BATCH], out[_BATCH:2 * _BATCH], out[2 * _BATCH:])
